# Initial kernel scaffold; baseline (speedup 1.0000x reference)
import jax
import jax.numpy as jnp
from jax.experimental import pallas as pl


def kernel(n_feat, edge_index, W_self, W_neigh, b_sage, W1, b1, W2, b2):
    raise NotImplementedError("write your pallas kernel here")


# speedup vs baseline: 3.3230x; 3.3230x over previous
import functools
import jax
import jax.numpy as jnp
from jax import lax
from jax.experimental import pallas as pl
from jax.experimental.pallas import tpu as pltpu
from jax.experimental.pallas import tpu_sc as plsc

N = 10000
E = 320000
D_IN = 128
D_H = 256
D_OUT = 128

NC = 2
NS = 16
LANES = 16

NPAD = 10240
STRIPE = NPAD // NS
CB = 128
CH = 158
EPAD = NS * CH * CB
CHB = 80
EPADB = NC * NS * CHB * CB


def _mesh():
  return plsc.VectorSubcoreMesh(
      core_axis_name="c", subcore_axis_name="s", num_cores=NC,
      num_subcores=NS)


def _unpack_edges(packed, didx, nrows):
  def row(r, _):
    for g in range(CB // LANES):
      p = packed[r, pl.ds(g * LANES, LANES)]
      didx[r, pl.ds(g * LANES, LANES)] = jnp.bitwise_and(p, 16383)
      packed[r, pl.ds(g * LANES, LANES)] = jnp.right_shift(p, 14)
    return 0

  lax.fori_loop(0, nrows, row, 0)


def _zero_stripe(buf2d, zvec, shared2d, shared1ds, s):
  rows = buf2d.shape[0]

  def zrow(i, _):
    for l in range(buf2d.shape[1] // LANES):
      buf2d[i, pl.ds(l * LANES, LANES)] = jnp.zeros((LANES,), jnp.float32)
    return 0

  lax.fori_loop(0, rows, zrow, 0)
  for l in range(zvec.shape[0] // LANES):
    zvec[pl.ds(l * LANES, LANES)] = jnp.zeros((LANES,), jnp.float32)
  for r in range(STRIPE // rows):
    pltpu.sync_copy(buf2d, shared2d.at[pl.ds(s * STRIPE + r * rows, rows)])
  for sh1 in shared1ds:
    pltpu.sync_copy(zvec, sh1.at[pl.ds(s * STRIPE, STRIPE)])


def _sc_layer1(edges_p, nf_pad):
  grid_out = (
      jax.ShapeDtypeStruct((NC, NPAD, D_IN), jnp.float32),
      jax.ShapeDtypeStruct((NC, NPAD), jnp.float32),
      jax.ShapeDtypeStruct((NC, NPAD), jnp.float32),
  )

  @functools.partial(
      pl.kernel,
      out_type=grid_out,
      mesh=_mesh(),
      scratch_types=[
          pltpu.VMEM((CHB, CB), jnp.int32),
          pltpu.VMEM((CHB, CB), jnp.int32),
          pltpu.VMEM((CB, D_IN), jnp.float32),
          pltpu.VMEM((CB,), jnp.float32),
          pltpu.VMEM((STRIPE,), jnp.float32),
          pltpu.VMEM_SHARED((NPAD, D_IN), jnp.float32),
          pltpu.VMEM_SHARED((NPAD,), jnp.float32),
          pltpu.VMEM_SHARED((NPAD,), jnp.float32),
          pltpu.SemaphoreType.DMA,
      ],
  )
  def k(edges_hbm, nf_hbm, neigh_hbm, di_hbm, do_hbm,
        sidx, didx, buf0, ones_v, zvec, acc, hin, hout, sem0):
    c = lax.axis_index("c")
    s = lax.axis_index("s")
    w = c * NS + s

    _zero_stripe(buf0, zvec, acc, [hin, hout], s)

    pltpu.sync_copy(edges_hbm.at[w], sidx)
    _unpack_edges(sidx, didx, CHB)
    for i in range(CB // LANES):
      ones_v[pl.ds(i * LANES, LANES)] = jnp.ones((LANES,), jnp.float32)

    plsc.subcore_barrier()

    def body(j, _):
      pltpu.async_copy(nf_hbm.at[sidx.at[j]], buf0, sem0).wait()
      pltpu.sync_copy(buf0, acc.at[didx.at[j]], add=True)
      pltpu.sync_copy(ones_v, hin.at[didx.at[j]], add=True)
      pltpu.sync_copy(ones_v, hout.at[sidx.at[j]], add=True)
      return 0

    lax.fori_loop(0, CHB, body, 0)

    plsc.subcore_barrier()

    rows = pl.ds(s * STRIPE, STRIPE)

    pltpu.sync_copy(acc.at[rows], neigh_hbm.at[c].at[rows])
    pltpu.sync_copy(hin.at[rows], di_hbm.at[c].at[rows])
    pltpu.sync_copy(hout.at[rows], do_hbm.at[c].at[rows])

  return k(edges_p, nf_pad)


def _sc_layer2_half(edges_p, xs_half, norm_dst):
  DH2 = D_H // 2
  grid_out = (
      jax.ShapeDtypeStruct((NC, NPAD, DH2), jnp.float32),
      jax.ShapeDtypeStruct((NC, NPAD), jnp.float32),
  )

  @functools.partial(
      pl.kernel,
      out_type=grid_out,
      mesh=_mesh(),
      scratch_types=[
          pltpu.VMEM((CHB, CB), jnp.int32),
          pltpu.VMEM((CHB, CB), jnp.int32),
          pltpu.VMEM((CB, DH2), jnp.float32),
          pltpu.VMEM((CB,), jnp.float32),
          pltpu.VMEM((STRIPE,), jnp.float32),
          pltpu.VMEM_SHARED((NPAD, DH2), jnp.float32),
          pltpu.VMEM_SHARED((NPAD,), jnp.float32),
          pltpu.SemaphoreType.DMA,
          pltpu.SemaphoreType.DMA,
      ],
  )
  def k(edges_hbm, xs_hbm, nd_hbm, agg_hbm, sp_hbm,
        sidx, didx, buf0, vals_v, zvec, acc, shist, sem0, sem1):
    c = lax.axis_index("c")
    s = lax.axis_index("s")
    w = c * NS + s

    _zero_stripe(buf0, zvec, acc, [shist], s)
    pltpu.sync_copy(edges_hbm.at[w], sidx)
    _unpack_edges(sidx, didx, CHB)

    plsc.subcore_barrier()

    def body(j, _):
      pltpu.async_copy(xs_hbm.at[sidx.at[j]], buf0, sem0).wait()
      pltpu.sync_copy(buf0, acc.at[didx.at[j]], add=True)
      pltpu.async_copy(nd_hbm.at[didx.at[j]], vals_v, sem1).wait()
      pltpu.sync_copy(vals_v, shist.at[sidx.at[j]], add=True)
      return 0

    lax.fori_loop(0, CHB, body, 0)

    plsc.subcore_barrier()

    rows = pl.ds(s * STRIPE, STRIPE)
    pltpu.sync_copy(acc.at[rows], agg_hbm.at[c].at[rows])
    pltpu.sync_copy(shist.at[rows], sp_hbm.at[c].at[rows])

  return k(edges_p, xs_half, norm_dst)


BM = 640


def _tc_sage(nf_pad, neigh, deg_in, deg_out, W_self, W_neigh, b_sage):
  nsteps = NPAD // BM

  def body(nf_ref, nb_ref, di_ref, do_ref,
           ws_ref, wn_ref, b_ref, xs0_ref, xs1_ref, nd_ref, ns_ref):
    i = pl.program_id(0)
    x = nf_ref[...]
    neigh = nb_ref[0] + nb_ref[1]
    di = di_ref[0] + di_ref[1]
    do = do_ref[0] + do_ref[1]
    inv_deg = 1.0 / jnp.maximum(di, 1.0)
    h_neigh = neigh * inv_deg
    h1 = (jnp.dot(x, ws_ref[...], preferred_element_type=jnp.float32)
          + b_ref[...]
          + jnp.dot(h_neigh, wn_ref[...], preferred_element_type=jnp.float32))
    h1 = jnp.where(h1 >= 0, h1, 0.01 * h1)
    rows = i * BM + lax.broadcasted_iota(jnp.int32, (BM, 1), 0)
    valid = rows < N
    norm_src = jnp.where(valid, lax.rsqrt(jnp.maximum(do, 1.0)), 0.0)
    norm_dst = jnp.where(valid, lax.rsqrt(jnp.maximum(di, 1.0)), 0.0)
    xs = h1 * norm_src
    xs0_ref[...] = xs[:, :D_H // 2]
    xs1_ref[...] = xs[:, D_H // 2:]
    nd_ref[...] = norm_dst
    ns_ref[...] = norm_src

  return pl.pallas_call(
      body,
      grid=(nsteps,),
      in_specs=[
          pl.BlockSpec((BM, D_IN), lambda i: (i, 0)),
          pl.BlockSpec((NC, BM, D_IN), lambda i: (0, i, 0)),
          pl.BlockSpec((NC, BM, 1), lambda i: (0, i, 0)),
          pl.BlockSpec((NC, BM, 1), lambda i: (0, i, 0)),
          pl.BlockSpec((D_IN, D_H), lambda i: (0, 0)),
          pl.BlockSpec((D_IN, D_H), lambda i: (0, 0)),
          pl.BlockSpec((1, D_H), lambda i: (0, 0)),
      ],
      out_specs=[
          pl.BlockSpec((BM, D_H // 2), lambda i: (i, 0)),
          pl.BlockSpec((BM, D_H // 2), lambda i: (i, 0)),
          pl.BlockSpec((BM, 1), lambda i: (i, 0)),
          pl.BlockSpec((BM, 1), lambda i: (i, 0)),
      ],
      out_shape=[
          jax.ShapeDtypeStruct((NPAD, D_H // 2), jnp.float32),
          jax.ShapeDtypeStruct((NPAD, D_H // 2), jnp.float32),
          jax.ShapeDtypeStruct((NPAD, 1), jnp.float32),
          jax.ShapeDtypeStruct((NPAD, 1), jnp.float32),
      ],
  )(nf_pad, neigh, deg_in, deg_out, W_self, W_neigh, b_sage)


def _tc_out(aggA, aggB, norm_dst, norm_src, sA, sB, W1, b1, W2, b2):
  nsteps = NPAD // BM

  def body(aA_ref, aB_ref, nd_ref, ns_ref, sA_ref, sB_ref,
           w1_ref, b1_ref, w2_ref, b2_ref, out_ref, acc_ref):
    i = pl.program_id(0)

    @pl.when(i == 0)
    def _():
      acc_ref[...] = jnp.zeros_like(acc_ref)

    agg = jnp.concatenate([aA_ref[0] + aA_ref[1], aB_ref[0] + aB_ref[1]],
                          axis=1) * nd_ref[...]
    h2 = jnp.dot(agg, w1_ref[...], preferred_element_type=jnp.float32) \
        + b1_ref[...]
    h2 = jnp.where(h2 >= 0, h2, 0.01 * h2)
    cw = ns_ref[...] * (0.5 * (sA_ref[0] + sA_ref[1] + sB_ref[0] + sB_ref[1]))
    acc_ref[...] += jnp.sum(h2 * cw, axis=0, keepdims=True)

    @pl.when(i == nsteps - 1)
    def _():
      mean_agg = acc_ref[...] * (1.0 / N)
      out_ref[...] = jnp.dot(mean_agg, w2_ref[...],
                             preferred_element_type=jnp.float32) + b2_ref[...]

  return pl.pallas_call(
      body,
      grid=(nsteps,),
      in_specs=[
          pl.BlockSpec((NC, BM, D_H // 2), lambda i: (0, i, 0)),
          pl.BlockSpec((NC, BM, D_H // 2), lambda i: (0, i, 0)),
          pl.BlockSpec((BM, 1), lambda i: (i, 0)),
          pl.BlockSpec((BM, 1), lambda i: (i, 0)),
          pl.BlockSpec((NC, BM, 1), lambda i: (0, i, 0)),
          pl.BlockSpec((NC, BM, 1), lambda i: (0, i, 0)),
          pl.BlockSpec((D_H, D_H), lambda i: (0, 0)),
          pl.BlockSpec((1, D_H), lambda i: (0, 0)),
          pl.BlockSpec((D_H, D_OUT), lambda i: (0, 0)),
          pl.BlockSpec((1, D_OUT), lambda i: (0, 0)),
      ],
      out_specs=pl.BlockSpec((1, D_OUT), lambda i: (0, 0)),
      out_shape=jax.ShapeDtypeStruct((1, D_OUT), jnp.float32),
      scratch_shapes=[pltpu.VMEM((1, D_H), jnp.float32)],
  )(aggA, aggB, norm_dst, norm_src, sA, sB, W1, b1, W2, b2)


def kernel(n_feat, edge_index, W_self, W_neigh, b_sage, W1, b1, W2, b2):
  f32 = jnp.float32
  src = edge_index[0]
  dst = edge_index[1]

  packed = jnp.left_shift(src, 14) | dst
  padv = jnp.full((), (N << 14) | N, jnp.int32)
  edges_pb = jnp.full((EPADB,), padv, jnp.int32).at[:E].set(packed) \
      .reshape(NC * NS, CHB, CB)

  nf_pad = jnp.zeros((NPAD, D_IN), f32).at[:N].set(n_feat)

  neigh, deg_in, deg_out = _sc_layer1(edges_pb, nf_pad)

  xs0, xs1, norm_dst, norm_src = _tc_sage(
      nf_pad, neigh, deg_in.reshape(NC, NPAD, 1), deg_out.reshape(NC, NPAD, 1),
      W_self, W_neigh, b_sage.reshape(1, D_H))

  nd1 = norm_dst.reshape(NPAD)
  aggA, sA = _sc_layer2_half(edges_pb, xs0, nd1)
  aggB, sB = _sc_layer2_half(edges_pb, xs1, nd1)

  out = _tc_out(aggA, aggB, norm_dst, norm_src,
                sA.reshape(NC, NPAD, 1), sB.reshape(NC, NPAD, 1),
                W1, b1.reshape(1, D_H), W2, b2.reshape(1, D_OUT))
  return out


# speedup vs baseline: 3.6697x; 1.1043x over previous
import functools
import jax
import jax.numpy as jnp
from jax import lax
from jax.experimental import pallas as pl
from jax.experimental.pallas import tpu as pltpu
from jax.experimental.pallas import tpu_sc as plsc

N = 10000
E = 320000
D_IN = 128
D_H = 256
D_OUT = 128

NC = 2
NS = 16
LANES = 16

NPAD = 10240
STRIPE = NPAD // NS
CB = 128
CHB = 80
EPADB = NC * NS * CHB * CB


def _mesh():
  return plsc.VectorSubcoreMesh(
      core_axis_name="c", subcore_axis_name="s", num_cores=NC,
      num_subcores=NS)


def _unpack_edges(packed, didx, nrows):
  def row(r, _):
    for g in range(CB // LANES):
      p = packed[r, pl.ds(g * LANES, LANES)]
      didx[r, pl.ds(g * LANES, LANES)] = jnp.bitwise_and(p, 16383)
      packed[r, pl.ds(g * LANES, LANES)] = jnp.right_shift(p, 14)
    return 0

  lax.fori_loop(0, nrows, row, 0)


def _zero_stripe(buf2d, zvec, shared2d, shared1ds, s):
  rows = buf2d.shape[0]

  def zrow(i, _):
    for l in range(buf2d.shape[1] // LANES):
      buf2d[i, pl.ds(l * LANES, LANES)] = jnp.zeros((LANES,), jnp.float32)
    return 0

  lax.fori_loop(0, rows, zrow, 0)
  for l in range(zvec.shape[0] // LANES):
    zvec[pl.ds(l * LANES, LANES)] = jnp.zeros((LANES,), jnp.float32)
  for r in range(STRIPE // rows):
    pltpu.sync_copy(buf2d, shared2d.at[pl.ds(s * STRIPE + r * rows, rows)])
  for sh1 in shared1ds:
    pltpu.sync_copy(zvec, sh1.at[pl.ds(s * STRIPE, STRIPE)])


def _sc_layer1(edges_p, nf_pad):
  grid_out = (
      jax.ShapeDtypeStruct((NC, NPAD, D_IN), jnp.float32),
      jax.ShapeDtypeStruct((NC, NPAD), jnp.float32),
      jax.ShapeDtypeStruct((NC, NPAD), jnp.float32),
  )

  @functools.partial(
      pl.kernel,
      out_type=grid_out,
      mesh=_mesh(),
      scratch_types=[
          pltpu.VMEM((CHB, CB), jnp.int32),
          pltpu.VMEM((CHB, CB), jnp.int32),
          pltpu.VMEM((CB, D_IN), jnp.float32),
          pltpu.VMEM((CB,), jnp.float32),
          pltpu.VMEM((STRIPE,), jnp.float32),
          pltpu.VMEM_SHARED((NPAD, D_IN), jnp.float32),
          pltpu.VMEM_SHARED((NPAD,), jnp.float32),
          pltpu.VMEM_SHARED((NPAD,), jnp.float32),
          pltpu.SemaphoreType.DMA,
      ],
  )
  def k(edges_hbm, nf_hbm, neigh_hbm, di_hbm, do_hbm,
        sidx, didx, buf0, ones_v, zvec, acc, hin, hout, sem0):
    c = lax.axis_index("c")
    s = lax.axis_index("s")
    w = c * NS + s

    _zero_stripe(buf0, zvec, acc, [hin, hout], s)

    pltpu.sync_copy(edges_hbm.at[w], sidx)
    _unpack_edges(sidx, didx, CHB)
    for i in range(CB // LANES):
      ones_v[pl.ds(i * LANES, LANES)] = jnp.ones((LANES,), jnp.float32)

    plsc.subcore_barrier()

    def body(j, _):
      pltpu.async_copy(nf_hbm.at[sidx.at[j]], buf0, sem0).wait()
      pltpu.sync_copy(buf0, acc.at[didx.at[j]], add=True)
      pltpu.sync_copy(ones_v, hin.at[didx.at[j]], add=True)
      pltpu.sync_copy(ones_v, hout.at[sidx.at[j]], add=True)
      return 0

    lax.fori_loop(0, CHB, body, 0)

    plsc.subcore_barrier()

    rows = pl.ds(s * STRIPE, STRIPE)

    pltpu.sync_copy(acc.at[rows], neigh_hbm.at[c].at[rows])
    pltpu.sync_copy(hin.at[rows], di_hbm.at[c].at[rows])
    pltpu.sync_copy(hout.at[rows], do_hbm.at[c].at[rows])

  return k(edges_p, nf_pad)


def _sc_layer2(edges_p, xs0, xs1, norm_dst):
  DH2 = D_H // 2
  grid_out = (
      jax.ShapeDtypeStruct((NC, NPAD, DH2), jnp.float32),
      jax.ShapeDtypeStruct((NC, NPAD, DH2), jnp.float32),
      jax.ShapeDtypeStruct((NC, NPAD), jnp.float32),
  )

  @functools.partial(
      pl.kernel,
      out_type=grid_out,
      mesh=_mesh(),
      scratch_types=[
          pltpu.VMEM((CHB, CB), jnp.int32),
          pltpu.VMEM((CHB, CB), jnp.int32),
          pltpu.VMEM((CB, DH2), jnp.float32),
          pltpu.VMEM((CB,), jnp.float32),
          pltpu.VMEM((STRIPE,), jnp.float32),
          pltpu.VMEM_SHARED((NPAD, DH2), jnp.float32),
          pltpu.VMEM_SHARED((NPAD,), jnp.float32),
          pltpu.SemaphoreType.DMA,
      ],
  )
  def k(edges_hbm, xs0_hbm, xs1_hbm, nd_hbm, aggA_hbm, aggB_hbm, sp_hbm,
        sidx, didx, buf0, vals0, zvec, acc, shist, sem0):
    c = lax.axis_index("c")
    s = lax.axis_index("s")
    w = c * NS + s
    rows = pl.ds(s * STRIPE, STRIPE)

    _zero_stripe(buf0, zvec, acc, [shist], s)
    pltpu.sync_copy(edges_hbm.at[w], sidx)
    _unpack_edges(sidx, didx, CHB)

    plsc.subcore_barrier()

    def feature_phase(xs_hbm):
      def body(j, _):
        pltpu.async_copy(xs_hbm.at[sidx.at[j]], buf0, sem0).wait()
        pltpu.sync_copy(buf0, acc.at[didx.at[j]], add=True)
        return 0

      lax.fori_loop(0, CHB, body, 0)

    feature_phase(xs0_hbm)

    def sbody(j, _):
      pltpu.async_copy(nd_hbm.at[didx.at[j]], vals0, sem0).wait()
      pltpu.sync_copy(vals0, shist.at[sidx.at[j]], add=True)
      return 0

    lax.fori_loop(0, CHB, sbody, 0)
    plsc.subcore_barrier()

    pltpu.sync_copy(acc.at[rows], aggA_hbm.at[c].at[rows])
    pltpu.sync_copy(shist.at[rows], sp_hbm.at[c].at[rows])
    _zero_stripe(buf0, zvec, acc, [], s)
    plsc.subcore_barrier()

    feature_phase(xs1_hbm)
    plsc.subcore_barrier()

    pltpu.sync_copy(acc.at[rows], aggB_hbm.at[c].at[rows])

  return k(edges_p, xs0, xs1, norm_dst)


BM = 640


def _tc_sage(nf_pad, neigh, deg_in, deg_out, W_self, W_neigh, b_sage):
  nsteps = NPAD // BM

  def body(nf_ref, nb_ref, di_ref, do_ref,
           ws_ref, wn_ref, b_ref, xs0_ref, xs1_ref, nd_ref, ns_ref):
    i = pl.program_id(0)
    x = nf_ref[...]
    neigh = nb_ref[0] + nb_ref[1]
    di = di_ref[0] + di_ref[1]
    do = do_ref[0] + do_ref[1]
    inv_deg = 1.0 / jnp.maximum(di, 1.0)
    h_neigh = neigh * inv_deg
    h1 = (jnp.dot(x, ws_ref[...], preferred_element_type=jnp.float32)
          + b_ref[...]
          + jnp.dot(h_neigh, wn_ref[...], preferred_element_type=jnp.float32))
    h1 = jnp.where(h1 >= 0, h1, 0.01 * h1)
    rows = i * BM + lax.broadcasted_iota(jnp.int32, (BM, 1), 0)
    valid = rows < N
    norm_src = jnp.where(valid, lax.rsqrt(jnp.maximum(do, 1.0)), 0.0)
    norm_dst = jnp.where(valid, lax.rsqrt(jnp.maximum(di, 1.0)), 0.0)
    xs = h1 * norm_src
    xs0_ref[...] = xs[:, :D_H // 2]
    xs1_ref[...] = xs[:, D_H // 2:]
    nd_ref[...] = norm_dst
    ns_ref[...] = norm_src

  return pl.pallas_call(
      body,
      grid=(nsteps,),
      in_specs=[
          pl.BlockSpec((BM, D_IN), lambda i: (i, 0)),
          pl.BlockSpec((NC, BM, D_IN), lambda i: (0, i, 0)),
          pl.BlockSpec((NC, BM, 1), lambda i: (0, i, 0)),
          pl.BlockSpec((NC, BM, 1), lambda i: (0, i, 0)),
          pl.BlockSpec((D_IN, D_H), lambda i: (0, 0)),
          pl.BlockSpec((D_IN, D_H), lambda i: (0, 0)),
          pl.BlockSpec((1, D_H), lambda i: (0, 0)),
      ],
      out_specs=[
          pl.BlockSpec((BM, D_H // 2), lambda i: (i, 0)),
          pl.BlockSpec((BM, D_H // 2), lambda i: (i, 0)),
          pl.BlockSpec((BM, 1), lambda i: (i, 0)),
          pl.BlockSpec((BM, 1), lambda i: (i, 0)),
      ],
      out_shape=[
          jax.ShapeDtypeStruct((NPAD, D_H // 2), jnp.float32),
          jax.ShapeDtypeStruct((NPAD, D_H // 2), jnp.float32),
          jax.ShapeDtypeStruct((NPAD, 1), jnp.float32),
          jax.ShapeDtypeStruct((NPAD, 1), jnp.float32),
      ],
  )(nf_pad, neigh, deg_in, deg_out, W_self, W_neigh, b_sage)


def _tc_out(aggA, aggB, norm_dst, norm_src, s_p, W1, b1, W2, b2):
  nsteps = NPAD // BM

  def body(aA_ref, aB_ref, nd_ref, ns_ref, s_ref,
           w1_ref, b1_ref, w2_ref, b2_ref, out_ref, acc_ref):
    i = pl.program_id(0)

    @pl.when(i == 0)
    def _():
      acc_ref[...] = jnp.zeros_like(acc_ref)

    agg = jnp.concatenate([aA_ref[0] + aA_ref[1], aB_ref[0] + aB_ref[1]],
                          axis=1) * nd_ref[...]
    h2 = jnp.dot(agg, w1_ref[...], preferred_element_type=jnp.float32) \
        + b1_ref[...]
    h2 = jnp.where(h2 >= 0, h2, 0.01 * h2)
    cw = ns_ref[...] * (s_ref[0] + s_ref[1])
    acc_ref[...] += jnp.sum(h2 * cw, axis=0, keepdims=True)

    @pl.when(i == nsteps - 1)
    def _():
      mean_agg = acc_ref[...] * (1.0 / N)
      out_ref[...] = jnp.dot(mean_agg, w2_ref[...],
                             preferred_element_type=jnp.float32) + b2_ref[...]

  return pl.pallas_call(
      body,
      grid=(nsteps,),
      in_specs=[
          pl.BlockSpec((NC, BM, D_H // 2), lambda i: (0, i, 0)),
          pl.BlockSpec((NC, BM, D_H // 2), lambda i: (0, i, 0)),
          pl.BlockSpec((BM, 1), lambda i: (i, 0)),
          pl.BlockSpec((BM, 1), lambda i: (i, 0)),
          pl.BlockSpec((NC, BM, 1), lambda i: (0, i, 0)),
          pl.BlockSpec((D_H, D_H), lambda i: (0, 0)),
          pl.BlockSpec((1, D_H), lambda i: (0, 0)),
          pl.BlockSpec((D_H, D_OUT), lambda i: (0, 0)),
          pl.BlockSpec((1, D_OUT), lambda i: (0, 0)),
      ],
      out_specs=pl.BlockSpec((1, D_OUT), lambda i: (0, 0)),
      out_shape=jax.ShapeDtypeStruct((1, D_OUT), jnp.float32),
      scratch_shapes=[pltpu.VMEM((1, D_H), jnp.float32)],
  )(aggA, aggB, norm_dst, norm_src, s_p, W1, b1, W2, b2)


def kernel(n_feat, edge_index, W_self, W_neigh, b_sage, W1, b1, W2, b2):
  f32 = jnp.float32
  src = edge_index[0]
  dst = edge_index[1]

  packed = jnp.left_shift(src, 14) | dst
  padv = jnp.full((), (N << 14) | N, jnp.int32)
  edges_pb = jnp.full((EPADB,), padv, jnp.int32).at[:E].set(packed) \
      .reshape(NC * NS, CHB, CB)

  nf_pad = jnp.zeros((NPAD, D_IN), f32).at[:N].set(n_feat)

  neigh, deg_in, deg_out = _sc_layer1(edges_pb, nf_pad)

  xs0, xs1, norm_dst, norm_src = _tc_sage(
      nf_pad, neigh, deg_in.reshape(NC, NPAD, 1), deg_out.reshape(NC, NPAD, 1),
      W_self, W_neigh, b_sage.reshape(1, D_H))

  aggA, aggB, s_p = _sc_layer2(edges_pb, xs0, xs1, norm_dst.reshape(NPAD))

  out = _tc_out(aggA, aggB, norm_dst, norm_src, s_p.reshape(NC, NPAD, 1),
                W1, b1.reshape(1, D_H), W2, b2.reshape(1, D_OUT))
  return out


# speedup vs baseline: 3.8842x; 1.0585x over previous
import functools
import jax
import jax.numpy as jnp
from jax import lax
from jax.experimental import pallas as pl
from jax.experimental.pallas import tpu as pltpu
from jax.experimental.pallas import tpu_sc as plsc

N = 10000
E = 320000
D_IN = 128
D_H = 256
D_OUT = 128

NC = 2
NS = 16
LANES = 16

NPAD = 10240
STRIPE = NPAD // NS
CB = 128
CHB = 80
EPADB = NC * NS * CHB * CB


def _mesh():
  return plsc.VectorSubcoreMesh(
      core_axis_name="c", subcore_axis_name="s", num_cores=NC,
      num_subcores=NS)


def _unpack_edges(packed, didx, nrows):
  def row(r, _):
    for g in range(CB // LANES):
      p = packed[r, pl.ds(g * LANES, LANES)]
      didx[r, pl.ds(g * LANES, LANES)] = jnp.bitwise_and(p, 16383)
      packed[r, pl.ds(g * LANES, LANES)] = jnp.right_shift(p, 14)
    return 0

  lax.fori_loop(0, nrows, row, 0)


def _zero_stripe(buf2d, zvec, shared2d, shared1ds, s):
  rows = buf2d.shape[0]

  def zrow(i, _):
    for l in range(buf2d.shape[1] // LANES):
      buf2d[i, pl.ds(l * LANES, LANES)] = jnp.zeros((LANES,), jnp.float32)
    return 0

  lax.fori_loop(0, rows, zrow, 0)
  for l in range(zvec.shape[0] // LANES):
    zvec[pl.ds(l * LANES, LANES)] = jnp.zeros((LANES,), jnp.float32)
  for r in range(STRIPE // rows):
    pltpu.sync_copy(buf2d, shared2d.at[pl.ds(s * STRIPE + r * rows, rows)])
  for sh1 in shared1ds:
    pltpu.sync_copy(zvec, sh1.at[pl.ds(s * STRIPE, STRIPE)])


def _sc_layer1(edges_p, nf_pad):
  grid_out = (
      jax.ShapeDtypeStruct((NC, NPAD, D_IN), jnp.float32),
      jax.ShapeDtypeStruct((NC, NPAD), jnp.float32),
      jax.ShapeDtypeStruct((NC, NPAD), jnp.float32),
  )

  @functools.partial(
      pl.kernel,
      out_type=grid_out,
      mesh=_mesh(),
      scratch_types=[
          pltpu.VMEM((CHB, CB), jnp.int32),
          pltpu.VMEM((CHB, CB), jnp.int32),
          pltpu.VMEM((CB, D_IN), jnp.float32),
          pltpu.VMEM((CB,), jnp.float32),
          pltpu.VMEM((STRIPE,), jnp.float32),
          pltpu.VMEM_SHARED((NPAD, D_IN), jnp.float32),
          pltpu.VMEM_SHARED((NPAD,), jnp.float32),
          pltpu.VMEM_SHARED((NPAD,), jnp.float32),
          pltpu.SemaphoreType.DMA,
      ],
  )
  def k(edges_hbm, nf_hbm, neigh_hbm, di_hbm, do_hbm,
        sidx, didx, buf0, ones_v, zvec, acc, hin, hout, sem0):
    c = lax.axis_index("c")
    s = lax.axis_index("s")
    w = c * NS + s

    _zero_stripe(buf0, zvec, acc, [hin, hout], s)

    pltpu.sync_copy(edges_hbm.at[w], sidx)
    _unpack_edges(sidx, didx, CHB)
    for i in range(CB // LANES):
      ones_v[pl.ds(i * LANES, LANES)] = jnp.ones((LANES,), jnp.float32)

    plsc.subcore_barrier()

    pltpu.async_copy(nf_hbm.at[sidx.at[0]], buf0, sem0)

    def body(j, _):
      pltpu.make_async_copy(nf_hbm.at[sidx.at[0]], buf0, sem0).wait()
      pltpu.sync_copy(buf0, acc.at[didx.at[j]], add=True)

      @pl.when(j + 1 < CHB)
      def _():
        pltpu.async_copy(nf_hbm.at[sidx.at[j + 1]], buf0, sem0)

      pltpu.sync_copy(ones_v, hin.at[didx.at[j]], add=True)
      pltpu.sync_copy(ones_v, hout.at[sidx.at[j]], add=True)
      return 0

    lax.fori_loop(0, CHB, body, 0)

    plsc.subcore_barrier()

    rows = pl.ds(s * STRIPE, STRIPE)

    pltpu.sync_copy(acc.at[rows], neigh_hbm.at[c].at[rows])
    pltpu.sync_copy(hin.at[rows], di_hbm.at[c].at[rows])
    pltpu.sync_copy(hout.at[rows], do_hbm.at[c].at[rows])

  return k(edges_p, nf_pad)


def _sc_layer2(edges_p, xs0, xs1, norm_dst):
  DH2 = D_H // 2
  grid_out = (
      jax.ShapeDtypeStruct((NC, NPAD, DH2), jnp.float32),
      jax.ShapeDtypeStruct((NC, NPAD, DH2), jnp.float32),
      jax.ShapeDtypeStruct((NC, NPAD), jnp.float32),
  )

  @functools.partial(
      pl.kernel,
      out_type=grid_out,
      mesh=_mesh(),
      scratch_types=[
          pltpu.VMEM((CHB, CB), jnp.int32),
          pltpu.VMEM((CHB, CB), jnp.int32),
          pltpu.VMEM((CB, DH2), jnp.float32),
          pltpu.VMEM((CB,), jnp.float32),
          pltpu.VMEM((STRIPE,), jnp.float32),
          pltpu.VMEM_SHARED((NPAD, DH2), jnp.float32),
          pltpu.VMEM_SHARED((NPAD,), jnp.float32),
          pltpu.SemaphoreType.DMA,
          pltpu.SemaphoreType.DMA,
      ],
  )
  def k(edges_hbm, xs0_hbm, xs1_hbm, nd_hbm, aggA_hbm, aggB_hbm, sp_hbm,
        sidx, didx, buf0, vals0, zvec, acc, shist, sem0, sem1):
    c = lax.axis_index("c")
    s = lax.axis_index("s")
    w = c * NS + s
    rows = pl.ds(s * STRIPE, STRIPE)

    _zero_stripe(buf0, zvec, acc, [shist], s)
    pltpu.sync_copy(edges_hbm.at[w], sidx)
    _unpack_edges(sidx, didx, CHB)

    plsc.subcore_barrier()

    def feature_phase(xs_hbm):
      def body(j, _):
        pltpu.async_copy(xs_hbm.at[sidx.at[j]], buf0, sem0).wait()
        pltpu.sync_copy(buf0, acc.at[didx.at[j]], add=True)
        return 0

      lax.fori_loop(0, CHB, body, 0)

    pltpu.async_copy(xs0_hbm.at[sidx.at[0]], buf0, sem0)

    def bodyA(j, _):
      pltpu.make_async_copy(xs0_hbm.at[sidx.at[0]], buf0, sem0).wait()
      pltpu.sync_copy(buf0, acc.at[didx.at[j]], add=True)

      @pl.when(j + 1 < CHB)
      def _():
        pltpu.async_copy(xs0_hbm.at[sidx.at[j + 1]], buf0, sem0)

      pltpu.async_copy(nd_hbm.at[didx.at[j]], vals0, sem1).wait()
      pltpu.sync_copy(vals0, shist.at[sidx.at[j]], add=True)
      return 0

    lax.fori_loop(0, CHB, bodyA, 0)
    plsc.subcore_barrier()

    pltpu.sync_copy(acc.at[rows], aggA_hbm.at[c].at[rows])
    pltpu.sync_copy(shist.at[rows], sp_hbm.at[c].at[rows])
    _zero_stripe(buf0, zvec, acc, [], s)
    plsc.subcore_barrier()

    feature_phase(xs1_hbm)
    plsc.subcore_barrier()

    pltpu.sync_copy(acc.at[rows], aggB_hbm.at[c].at[rows])

  return k(edges_p, xs0, xs1, norm_dst)


BM = 640


def _tc_sage(nf_pad, neigh, deg_in, deg_out, W_self, W_neigh, b_sage):
  nsteps = NPAD // BM

  def body(nf_ref, nb_ref, di_ref, do_ref,
           ws_ref, wn_ref, b_ref, xs0_ref, xs1_ref, nd_ref, ns_ref):
    i = pl.program_id(0)
    x = nf_ref[...]
    neigh = nb_ref[0] + nb_ref[1]
    di = di_ref[0] + di_ref[1]
    do = do_ref[0] + do_ref[1]
    inv_deg = 1.0 / jnp.maximum(di, 1.0)
    h_neigh = neigh * inv_deg
    h1 = (jnp.dot(x, ws_ref[...], preferred_element_type=jnp.float32)
          + b_ref[...]
          + jnp.dot(h_neigh, wn_ref[...], preferred_element_type=jnp.float32))
    h1 = jnp.where(h1 >= 0, h1, 0.01 * h1)
    rows = i * BM + lax.broadcasted_iota(jnp.int32, (BM, 1), 0)
    valid = rows < N
    norm_src = jnp.where(valid, lax.rsqrt(jnp.maximum(do, 1.0)), 0.0)
    norm_dst = jnp.where(valid, lax.rsqrt(jnp.maximum(di, 1.0)), 0.0)
    xs = h1 * norm_src
    xs0_ref[...] = xs[:, :D_H // 2]
    xs1_ref[...] = xs[:, D_H // 2:]
    nd_ref[...] = norm_dst
    ns_ref[...] = norm_src

  return pl.pallas_call(
      body,
      grid=(nsteps,),
      in_specs=[
          pl.BlockSpec((BM, D_IN), lambda i: (i, 0)),
          pl.BlockSpec((NC, BM, D_IN), lambda i: (0, i, 0)),
          pl.BlockSpec((NC, BM, 1), lambda i: (0, i, 0)),
          pl.BlockSpec((NC, BM, 1), lambda i: (0, i, 0)),
          pl.BlockSpec((D_IN, D_H), lambda i: (0, 0)),
          pl.BlockSpec((D_IN, D_H), lambda i: (0, 0)),
          pl.BlockSpec((1, D_H), lambda i: (0, 0)),
      ],
      out_specs=[
          pl.BlockSpec((BM, D_H // 2), lambda i: (i, 0)),
          pl.BlockSpec((BM, D_H // 2), lambda i: (i, 0)),
          pl.BlockSpec((BM, 1), lambda i: (i, 0)),
          pl.BlockSpec((BM, 1), lambda i: (i, 0)),
      ],
      out_shape=[
          jax.ShapeDtypeStruct((NPAD, D_H // 2), jnp.float32),
          jax.ShapeDtypeStruct((NPAD, D_H // 2), jnp.float32),
          jax.ShapeDtypeStruct((NPAD, 1), jnp.float32),
          jax.ShapeDtypeStruct((NPAD, 1), jnp.float32),
      ],
  )(nf_pad, neigh, deg_in, deg_out, W_self, W_neigh, b_sage)


def _tc_out(aggA, aggB, norm_dst, norm_src, s_p, W1, b1, W2, b2):
  nsteps = NPAD // BM

  def body(aA_ref, aB_ref, nd_ref, ns_ref, s_ref,
           w1_ref, b1_ref, w2_ref, b2_ref, out_ref, acc_ref):
    i = pl.program_id(0)

    @pl.when(i == 0)
    def _():
      acc_ref[...] = jnp.zeros_like(acc_ref)

    agg = jnp.concatenate([aA_ref[0] + aA_ref[1], aB_ref[0] + aB_ref[1]],
                          axis=1) * nd_ref[...]
    h2 = jnp.dot(agg, w1_ref[...], preferred_element_type=jnp.float32) \
        + b1_ref[...]
    h2 = jnp.where(h2 >= 0, h2, 0.01 * h2)
    cw = ns_ref[...] * (s_ref[0] + s_ref[1])
    acc_ref[...] += jnp.sum(h2 * cw, axis=0, keepdims=True)

    @pl.when(i == nsteps - 1)
    def _():
      mean_agg = acc_ref[...] * (1.0 / N)
      out_ref[...] = jnp.dot(mean_agg, w2_ref[...],
                             preferred_element_type=jnp.float32) + b2_ref[...]

  return pl.pallas_call(
      body,
      grid=(nsteps,),
      in_specs=[
          pl.BlockSpec((NC, BM, D_H // 2), lambda i: (0, i, 0)),
          pl.BlockSpec((NC, BM, D_H // 2), lambda i: (0, i, 0)),
          pl.BlockSpec((BM, 1), lambda i: (i, 0)),
          pl.BlockSpec((BM, 1), lambda i: (i, 0)),
          pl.BlockSpec((NC, BM, 1), lambda i: (0, i, 0)),
          pl.BlockSpec((D_H, D_H), lambda i: (0, 0)),
          pl.BlockSpec((1, D_H), lambda i: (0, 0)),
          pl.BlockSpec((D_H, D_OUT), lambda i: (0, 0)),
          pl.BlockSpec((1, D_OUT), lambda i: (0, 0)),
      ],
      out_specs=pl.BlockSpec((1, D_OUT), lambda i: (0, 0)),
      out_shape=jax.ShapeDtypeStruct((1, D_OUT), jnp.float32),
      scratch_shapes=[pltpu.VMEM((1, D_H), jnp.float32)],
  )(aggA, aggB, norm_dst, norm_src, s_p, W1, b1, W2, b2)


def kernel(n_feat, edge_index, W_self, W_neigh, b_sage, W1, b1, W2, b2):
  f32 = jnp.float32
  src = edge_index[0]
  dst = edge_index[1]

  packed = jnp.left_shift(src, 14) | dst
  padv = jnp.full((), (N << 14) | N, jnp.int32)
  edges_pb = jnp.full((EPADB,), padv, jnp.int32).at[:E].set(packed) \
      .reshape(NC * NS, CHB, CB)

  nf_pad = jnp.zeros((NPAD, D_IN), f32).at[:N].set(n_feat)

  neigh, deg_in, deg_out = _sc_layer1(edges_pb, nf_pad)

  xs0, xs1, norm_dst, norm_src = _tc_sage(
      nf_pad, neigh, deg_in.reshape(NC, NPAD, 1), deg_out.reshape(NC, NPAD, 1),
      W_self, W_neigh, b_sage.reshape(1, D_H))

  aggA, aggB, s_p = _sc_layer2(edges_pb, xs0, xs1, norm_dst.reshape(NPAD))

  out = _tc_out(aggA, aggB, norm_dst, norm_src, s_p.reshape(NC, NPAD, 1),
                W1, b1.reshape(1, D_H), W2, b2.reshape(1, D_OUT))
  return out
